# baseline (speedup 1.0000x reference)
import functools

import jax
import jax.numpy as jnp
from jax import lax
from jax.experimental import pallas as pl
from jax.experimental.pallas import tpu as pltpu
from jax.experimental.pallas import tpu_sc as plsc

VOCAB = 1000000
DIM = 64
SEQ = 200
BATCH = 4096
TOTAL = SEQ * BATCH

NC = 2
NS = 16
NW = NC * NS
B_PER_W = TOTAL // NW
G = 128
NG = B_PER_W // G

_mesh = plsc.VectorSubcoreMesh(core_axis_name="c", subcore_axis_name="s")


@functools.partial(
    pl.kernel,
    mesh=_mesh,
    out_type=jax.ShapeDtypeStruct((TOTAL, DIM), jnp.float32),
    scratch_types=[
        pltpu.VMEM((NG, G), jnp.int32),
        pltpu.VMEM((G, DIM), jnp.float32),
        pltpu.SemaphoreType.DMA,
    ],
    compiler_params=pltpu.CompilerParams(use_tc_tiling_on_sc=False),
)
def _sc_gather(table_hbm, idx_hbm, out_hbm, idx_v, rows_v, sem):
    wid = lax.axis_index("s") * NC + lax.axis_index("c")
    pltpu.sync_copy(idx_hbm.at[pl.ds(wid * NG, NG)], idx_v)
    base = wid * B_PER_W

    def body(j, carry):
        pltpu.async_copy(table_hbm.at[idx_v.at[j]], rows_v, sem).wait()
        pltpu.sync_copy(rows_v, out_hbm.at[pl.ds(base + j * G, G)])
        return carry

    lax.fori_loop(0, NG, body, 0)


def kernel(src_input, word_lut):
    idx = src_input.reshape(NW * NG, G)
    out = _sc_gather(word_lut, idx)
    return out.reshape(SEQ, BATCH, DIM)


# speedup vs baseline: 1.1193x; 1.1193x over previous
import functools

import jax
import jax.numpy as jnp
from jax import lax
from jax.experimental import pallas as pl
from jax.experimental.pallas import tpu as pltpu
from jax.experimental.pallas import tpu_sc as plsc

VOCAB = 1000000
DIM = 64
SEQ = 200
BATCH = 4096
TOTAL = SEQ * BATCH

NC = 2
NS = 16
NW = NC * NS
B_PER_W = TOTAL // NW
G = 128
NG = B_PER_W // G
K = 4
CH = K * G
NI = B_PER_W // CH

_mesh = plsc.VectorSubcoreMesh(core_axis_name="c", subcore_axis_name="s")


@functools.partial(
    pl.kernel,
    mesh=_mesh,
    out_type=jax.ShapeDtypeStruct((TOTAL, DIM), jnp.float32),
    scratch_types=[
        pltpu.VMEM((NG, G), jnp.int32),
        pltpu.VMEM((CH, DIM), jnp.float32),
        pltpu.VMEM((CH, DIM), jnp.float32),
        pltpu.SemaphoreType.DMA,
        pltpu.SemaphoreType.DMA,
        pltpu.SemaphoreType.DMA,
    ],
    compiler_params=pltpu.CompilerParams(use_tc_tiling_on_sc=False),
)
def _sc_gather(table_hbm, idx_hbm, out_hbm, idx_v, rows0, rows1, gsem, wsem0, wsem1):
    wid = lax.axis_index("s") * NC + lax.axis_index("c")
    pltpu.sync_copy(idx_hbm.at[pl.ds(wid * NG, NG)], idx_v)
    base = wid * B_PER_W

    def fire_and_drain(j, rows_v):
        handles = [
            pltpu.async_copy(
                table_hbm.at[idx_v.at[j * K + b]],
                rows_v.at[pl.ds(b * G, G)],
                gsem,
            )
            for b in range(K)
        ]
        for h in handles:
            h.wait()

    def start_write(j, rows_v, wsem):
        pltpu.async_copy(rows_v, out_hbm.at[pl.ds(base + j * CH, CH)], wsem)

    def wait_write(rows_v, wsem):
        pltpu.make_async_copy(rows_v, out_hbm.at[pl.ds(base, CH)], wsem).wait()

    fire_and_drain(0, rows0)
    start_write(0, rows0, wsem0)
    fire_and_drain(1, rows1)
    start_write(1, rows1, wsem1)

    def body(c, carry):
        j0 = 2 * c
        wait_write(rows0, wsem0)
        fire_and_drain(j0, rows0)
        start_write(j0, rows0, wsem0)
        wait_write(rows1, wsem1)
        fire_and_drain(j0 + 1, rows1)
        start_write(j0 + 1, rows1, wsem1)
        return carry

    lax.fori_loop(1, NI // 2, body, 0)

    wait_write(rows0, wsem0)
    wait_write(rows1, wsem1)


def kernel(src_input, word_lut):
    idx = src_input.reshape(NW * NG, G)
    out = _sc_gather(word_lut, idx)
    return out.reshape(SEQ, BATCH, DIM)
